# baseline (speedup 1.0000x reference)
import functools

import jax
import jax.numpy as jnp
from jax import lax
from jax.experimental import pallas as pl
from jax.experimental.pallas import tpu as pltpu
from jax.experimental.pallas import tpu_sc as plsc

NC, NS = 2, 16
NW = NC * NS
CH = 128


def _gather_body(nch, x_hbm, idx_hbm, out_hbm, idx_v, rows_v, sem):
    c = lax.axis_index("c")
    s = lax.axis_index("s")
    wid = s * NC + c
    epw = nch * CH
    pltpu.sync_copy(idx_hbm.at[wid], idx_v)

    def fire(j, carry):
        pltpu.make_async_copy(
            x_hbm.at[idx_v.at[j]],
            rows_v.at[pl.ds(j * CH, CH)],
            sem,
        ).start()
        return carry

    def drain(j, carry):
        pltpu.make_async_copy(
            x_hbm.at[idx_v.at[0]],
            rows_v.at[pl.ds(0, CH)],
            sem,
        ).wait()
        return carry

    lax.fori_loop(0, nch, fire, 0)
    lax.fori_loop(0, nch, drain, 0)
    pltpu.sync_copy(rows_v, out_hbm.at[pl.ds(wid * epw, epw)])


def _sc_gather(x, idx_grouped, nin, nch):
    epw = nch * CH
    run = pl.kernel(
        functools.partial(_gather_body, nch),
        out_type=jax.ShapeDtypeStruct((NW * epw, nin), jnp.float32),
        mesh=plsc.VectorSubcoreMesh(core_axis_name="c", subcore_axis_name="s"),
        scratch_types=[
            pltpu.VMEM((nch, CH), jnp.int32),
            pltpu.VMEM((epw, nin), jnp.float32),
            pltpu.SemaphoreType.DMA,
        ],
        compiler_params=pltpu.CompilerParams(use_tc_tiling_on_sc=False),
    )
    return run(x, idx_grouped)


def _scatter_body(nch, n_acc, n_out, nout, msg_hbm, idx_hbm, zeros_hbm,
                  part_hbm, idx_v, rows_v, tmp_v, acc_sh, sem):
    c = lax.axis_index("c")
    s = lax.axis_index("s")
    wid = s * NC + c
    epw = nch * CH
    cnt = n_acc // NS

    pltpu.sync_copy(zeros_hbm.at[pl.ds(s * cnt, cnt)], tmp_v)
    pltpu.sync_copy(tmp_v, acc_sh.at[pl.ds(s * cnt, cnt)])

    pltpu.sync_copy(idx_hbm.at[wid], idx_v)
    pltpu.sync_copy(msg_hbm.at[pl.ds(wid * epw, epw)], rows_v)
    plsc.subcore_barrier()

    def fire(j, carry):
        pltpu.async_copy(
            rows_v.at[pl.ds(j * CH, CH)],
            acc_sh.at[idx_v.at[j]],
            sem,
            add=True,
        )
        return carry

    def drain(j, carry):
        pltpu.make_async_copy(
            rows_v.at[pl.ds(0, CH)],
            acc_sh.at[idx_v.at[0]],
            sem,
        ).wait()
        return carry

    lax.fori_loop(0, nch, fire, 0)
    lax.fori_loop(0, nch, drain, 0)
    plsc.subcore_barrier()

    ocnt = n_out // NS
    pltpu.sync_copy(acc_sh.at[pl.ds(s * ocnt, ocnt)], tmp_v.at[pl.ds(0, ocnt)])
    pltpu.sync_copy(tmp_v.at[pl.ds(0, ocnt)],
                    part_hbm.at[c].at[pl.ds(s * ocnt, ocnt)])


def _sc_scatter(msg, idx_grouped, zeros_acc, n_acc, n_out, nout, nch):
    epw = nch * CH
    run = pl.kernel(
        functools.partial(_scatter_body, nch, n_acc, n_out, nout),
        out_type=jax.ShapeDtypeStruct((NC, n_out, nout), jnp.float32),
        mesh=plsc.VectorSubcoreMesh(core_axis_name="c", subcore_axis_name="s"),
        scratch_types=[
            pltpu.VMEM((nch, CH), jnp.int32),
            pltpu.VMEM((epw, nout), jnp.float32),
            pltpu.VMEM((n_acc // NS, nout), jnp.float32),
            pltpu.VMEM_SHARED((n_acc, nout), jnp.float32),
            pltpu.SemaphoreType.DMA,
        ],
        compiler_params=pltpu.CompilerParams(use_tc_tiling_on_sc=False),
    )
    return run(msg, idx_grouped, zeros_acc)


def _edge_tc(be, nin, nout, ea_ref, xj_ref, w1_ref, w2_ref, rep_ref, sel_ref,
             msg_ref):
    g = 128 // nin
    bp = be // g
    ea = jnp.concatenate(
        [ea_ref[:, k * nin:(k + 1) * nin] for k in range(g)], axis=0)
    xj = jnp.concatenate(
        [xj_ref[:, k * nin:(k + 1) * nin] for k in range(g)], axis=0)
    a = jnp.dot(ea, w1_ref[...], preferred_element_type=jnp.float32)
    r = jnp.maximum(a, 0.0)
    h = jnp.dot(r.astype(jnp.bfloat16), w2_ref[...].astype(jnp.bfloat16),
                preferred_element_type=jnp.float32)
    xr = jnp.dot(xj, rep_ref[...], preferred_element_type=jnp.float32)
    msg = jnp.dot(xr * h, sel_ref[...], preferred_element_type=jnp.float32)
    msg_ref[...] = jnp.concatenate(
        [msg[k * bp:(k + 1) * bp, :] for k in range(g)], axis=1)


def _tc_edge(ea_pk, xj_pk, w1, w2, rep, sel, be, interpret=False):
    nin = w1.shape[0]
    hid = w2.shape[1]
    nout = sel.shape[1]
    e = ea_pk.shape[0] * 128 // nin
    grid = e // be
    bp = be * nin // 128
    bo = be * nout // 128
    return pl.pallas_call(
        functools.partial(_edge_tc, be, nin, nout),
        grid=(grid,),
        in_specs=[
            pl.BlockSpec((bp, 128), lambda i: (i, 0)),
            pl.BlockSpec((bp, 128), lambda i: (i, 0)),
            pl.BlockSpec((nin, hid), lambda i: (0, 0)),
            pl.BlockSpec((hid, hid), lambda i: (0, 0)),
            pl.BlockSpec((nin, hid), lambda i: (0, 0)),
            pl.BlockSpec((hid, nout), lambda i: (0, 0)),
        ],
        out_specs=pl.BlockSpec((bo, 128), lambda i: (i, 0)),
        out_shape=jax.ShapeDtypeStruct((e * nout // 128, 128), jnp.float32),
        interpret=interpret,
    )(ea_pk, xj_pk, w1, w2, rep, sel)


def _combine_tc(p_ref, x_ref, root_ref, bias_ref, out_ref):
    xr = jnp.dot(x_ref[...], root_ref[...], preferred_element_type=jnp.float32)
    out_ref[...] = p_ref[0] + p_ref[1] + xr + bias_ref[...]


def _tc_combine(parts, x, root, bias2d, bn, interpret=False):
    n, nin = x.shape
    nout = root.shape[1]
    grid = n // bn
    return pl.pallas_call(
        _combine_tc,
        grid=(grid,),
        in_specs=[
            pl.BlockSpec((NC, bn, nout), lambda i: (0, i, 0)),
            pl.BlockSpec((bn, nin), lambda i: (i, 0)),
            pl.BlockSpec((nin, nout), lambda i: (0, 0)),
            pl.BlockSpec((1, nout), lambda i: (0, 0)),
        ],
        out_specs=pl.BlockSpec((bn, nout), lambda i: (i, 0)),
        out_shape=jax.ShapeDtypeStruct((n, nout), jnp.float32),
        interpret=interpret,
    )(parts, x, root, bias2d)


def kernel(x, edge_index, edge_attr, mlp_w1, mlp_w2, root, bias):
    n, nin = x.shape
    e = edge_index.shape[1]
    hid = mlp_w1.shape[1]
    nout = root.shape[1]
    assert n % NS == 0

    nch = -(-e // (NW * CH))
    e_pad = NW * nch * CH
    src = edge_index[0]
    dst = edge_index[1]
    pad = e_pad - e
    if pad:
        src = jnp.concatenate([src, jnp.zeros((pad,), jnp.int32)])
        dst = jnp.concatenate([dst, jnp.full((pad,), n, jnp.int32)])
    src_g = src.reshape(NW, nch, CH)
    dst_g = dst.reshape(NW, nch, CH)
    ea_pk = edge_attr.reshape(e * nin // 128, 128)

    n_acc = -(-(n + 1) // NS) * NS
    zeros_acc = jnp.zeros((n_acc, nout), jnp.float32)

    ii = lax.broadcasted_iota(jnp.int32, (nin, hid), 0)
    cc = lax.broadcasted_iota(jnp.int32, (nin, hid), 1)
    rep = (cc // nout == ii).astype(jnp.float32)
    c2 = lax.broadcasted_iota(jnp.int32, (hid, nout), 0)
    oo = lax.broadcasted_iota(jnp.int32, (hid, nout), 1)
    sel = (c2 % nout == oo).astype(jnp.float32)

    xj = _sc_gather(x, src_g, nin, nch)
    xj_pk = xj.reshape(e_pad * nin // 128, 128)
    msg_pk = _tc_edge(ea_pk, xj_pk, mlp_w1, mlp_w2, rep, sel, be=3200)
    msg = msg_pk.reshape(e, nout)
    if pad:
        msg = jnp.concatenate([msg, jnp.zeros((pad, nout), jnp.float32)])
    parts = _sc_scatter(msg, dst_g, zeros_acc, n_acc, n, nout, nch)
    out = _tc_combine(parts, x, root, bias.reshape(1, nout), bn=2000)
    return out


# speedup vs baseline: 1.6070x; 1.6070x over previous
import functools

import jax
import jax.numpy as jnp
from jax import lax
from jax.experimental import pallas as pl
from jax.experimental.pallas import tpu as pltpu
from jax.experimental.pallas import tpu_sc as plsc

NC, NS = 2, 16
NW = NC * NS
CH = 128


def _gather_body(nch, x_hbm, idx_hbm, out_hbm, idx_v, rows_v, sem):
    c = lax.axis_index("c")
    s = lax.axis_index("s")
    wid = s * NC + c
    epw = nch * CH
    pltpu.sync_copy(idx_hbm.at[wid], idx_v)

    def fire(j, carry):
        pltpu.make_async_copy(
            x_hbm.at[idx_v.at[j]],
            rows_v.at[pl.ds(j * CH, CH)],
            sem,
        ).start()
        return carry

    def drain(j, carry):
        pltpu.make_async_copy(
            x_hbm.at[idx_v.at[0]],
            rows_v.at[pl.ds(0, CH)],
            sem,
        ).wait()
        return carry

    lax.fori_loop(0, nch, fire, 0)
    lax.fori_loop(0, nch, drain, 0)
    pltpu.sync_copy(rows_v, out_hbm.at[pl.ds(wid * epw, epw)])


def _sc_gather(x, idx_grouped, nin, nch):
    epw = nch * CH
    run = pl.kernel(
        functools.partial(_gather_body, nch),
        out_type=jax.ShapeDtypeStruct((NW * epw, nin), jnp.float32),
        mesh=plsc.VectorSubcoreMesh(core_axis_name="c", subcore_axis_name="s"),
        scratch_types=[
            pltpu.VMEM((nch, CH), jnp.int32),
            pltpu.VMEM((epw, nin), jnp.float32),
            pltpu.SemaphoreType.DMA,
        ],
        compiler_params=pltpu.CompilerParams(use_tc_tiling_on_sc=False),
    )
    return run(x, idx_grouped)


def _scatter_body(nch, n_acc, n_out, nout, msg_hbm, idx_hbm, zeros_hbm,
                  part_hbm, idx_v, rows_v, tmp_v, acc_sh, sem):
    c = lax.axis_index("c")
    s = lax.axis_index("s")
    wid = s * NC + c
    epw = nch * CH
    cnt = n_acc // NS

    pltpu.sync_copy(zeros_hbm.at[pl.ds(s * cnt, cnt)], tmp_v)
    pltpu.sync_copy(tmp_v, acc_sh.at[pl.ds(s * cnt, cnt)])

    pltpu.sync_copy(idx_hbm.at[wid], idx_v)
    pltpu.sync_copy(msg_hbm.at[pl.ds(wid * epw, epw)], rows_v)
    plsc.subcore_barrier()

    def fire(j, carry):
        pltpu.async_copy(
            rows_v.at[pl.ds(j * CH, CH)],
            acc_sh.at[idx_v.at[j]],
            sem,
            add=True,
        )
        return carry

    def drain(j, carry):
        pltpu.make_async_copy(
            rows_v.at[pl.ds(0, CH)],
            acc_sh.at[idx_v.at[0]],
            sem,
        ).wait()
        return carry

    lax.fori_loop(0, nch, fire, 0)
    lax.fori_loop(0, nch, drain, 0)
    plsc.subcore_barrier()

    ocnt = n_out // NS
    pltpu.sync_copy(acc_sh.at[pl.ds(s * ocnt, ocnt)], tmp_v.at[pl.ds(0, ocnt)])
    pltpu.sync_copy(tmp_v.at[pl.ds(0, ocnt)],
                    part_hbm.at[c].at[pl.ds(s * ocnt, ocnt)])


def _sc_scatter(msg, idx_grouped, zeros_acc, n_acc, n_out, nout, nch):
    epw = nch * CH
    run = pl.kernel(
        functools.partial(_scatter_body, nch, n_acc, n_out, nout),
        out_type=jax.ShapeDtypeStruct((NC, n_out, nout), jnp.float32),
        mesh=plsc.VectorSubcoreMesh(core_axis_name="c", subcore_axis_name="s"),
        scratch_types=[
            pltpu.VMEM((nch, CH), jnp.int32),
            pltpu.VMEM((epw, nout), jnp.float32),
            pltpu.VMEM((n_acc // NS, nout), jnp.float32),
            pltpu.VMEM_SHARED((n_acc, nout), jnp.float32),
            pltpu.SemaphoreType.DMA,
        ],
        compiler_params=pltpu.CompilerParams(use_tc_tiling_on_sc=False),
    )
    return run(msg, idx_grouped, zeros_acc)


def _edge_tc(be, nin, nout, ea_ref, xj_ref, w1_ref, w2_ref, rep_ref, sel_ref,
             msg_ref):
    g = 128 // nin
    bp = be // g
    ea = jnp.concatenate(
        [ea_ref[:, k * nin:(k + 1) * nin] for k in range(g)], axis=0)
    xj = jnp.concatenate(
        [xj_ref[:, k * nin:(k + 1) * nin] for k in range(g)], axis=0)
    a = jnp.dot(ea, w1_ref[...], preferred_element_type=jnp.float32)
    r = jnp.maximum(a, 0.0)
    h = jnp.dot(r.astype(jnp.bfloat16), w2_ref[...].astype(jnp.bfloat16),
                preferred_element_type=jnp.float32)
    xr = jnp.dot(xj, rep_ref[...], preferred_element_type=jnp.float32)
    msg = jnp.dot(xr * h, sel_ref[...], preferred_element_type=jnp.float32)
    msg_ref[...] = jnp.concatenate(
        [msg[k * bp:(k + 1) * bp, :] for k in range(g)], axis=1)


def _tc_edge(ea_pk, xj_pk, w1, w2, rep, sel, be, e_pad, interpret=False):
    nin = w1.shape[0]
    hid = w2.shape[1]
    nout = sel.shape[1]
    e = ea_pk.shape[0] * 128 // nin
    grid = e // be
    bp = be * nin // 128
    bo = be * nout // 128
    return pl.pallas_call(
        functools.partial(_edge_tc, be, nin, nout),
        grid=(grid,),
        in_specs=[
            pl.BlockSpec((bp, 128), lambda i: (i, 0)),
            pl.BlockSpec((bp, 128), lambda i: (i, 0)),
            pl.BlockSpec((nin, hid), lambda i: (0, 0)),
            pl.BlockSpec((hid, hid), lambda i: (0, 0)),
            pl.BlockSpec((nin, hid), lambda i: (0, 0)),
            pl.BlockSpec((hid, nout), lambda i: (0, 0)),
        ],
        out_specs=pl.BlockSpec((bo, 128), lambda i: (i, 0)),
        out_shape=jax.ShapeDtypeStruct((e_pad * nout // 128, 128),
                                       jnp.float32),
        interpret=interpret,
    )(ea_pk, xj_pk, w1, w2, rep, sel)


def _combine_tc(p_ref, x_ref, root_ref, bias_ref, out_ref):
    xr = jnp.dot(x_ref[...], root_ref[...], preferred_element_type=jnp.float32)
    out_ref[...] = p_ref[0] + p_ref[1] + xr + bias_ref[...]


def _tc_combine(parts, x, root, bias2d, bn, interpret=False):
    n, nin = x.shape
    nout = root.shape[1]
    grid = n // bn
    return pl.pallas_call(
        _combine_tc,
        grid=(grid,),
        in_specs=[
            pl.BlockSpec((NC, bn, nout), lambda i: (0, i, 0)),
            pl.BlockSpec((bn, nin), lambda i: (i, 0)),
            pl.BlockSpec((nin, nout), lambda i: (0, 0)),
            pl.BlockSpec((1, nout), lambda i: (0, 0)),
        ],
        out_specs=pl.BlockSpec((bn, nout), lambda i: (i, 0)),
        out_shape=jax.ShapeDtypeStruct((n, nout), jnp.float32),
        interpret=interpret,
    )(parts, x, root, bias2d)


def kernel(x, edge_index, edge_attr, mlp_w1, mlp_w2, root, bias):
    n, nin = x.shape
    e = edge_index.shape[1]
    hid = mlp_w1.shape[1]
    nout = root.shape[1]
    assert n % NS == 0

    nch = -(-e // (NW * CH))
    e_pad = NW * nch * CH
    src = edge_index[0]
    dst = edge_index[1]
    pad = e_pad - e
    if pad:
        src = jnp.concatenate([src, jnp.zeros((pad,), jnp.int32)])
        dst = jnp.concatenate([dst, jnp.full((pad,), n, jnp.int32)])
    src_g = src.reshape(NW, nch, CH)
    dst_g = dst.reshape(NW, nch, CH)
    ea_pk = edge_attr.reshape(e * nin // 128, 128)

    n_acc = -(-(n + 1) // NS) * NS
    zeros_acc = jnp.zeros((n_acc * nout // 128, 128),
                          jnp.float32).reshape(n_acc, nout)

    x_pk = lax.optimization_barrier(x.reshape(n * nin // 128, 128))
    x_lin = x_pk.reshape(n, nin)

    ii = lax.broadcasted_iota(jnp.int32, (nin, hid), 0)
    cc = lax.broadcasted_iota(jnp.int32, (nin, hid), 1)
    rep = (cc // nout == ii).astype(jnp.float32)
    c2 = lax.broadcasted_iota(jnp.int32, (hid, nout), 0)
    oo = lax.broadcasted_iota(jnp.int32, (hid, nout), 1)
    sel = (c2 % nout == oo).astype(jnp.float32)

    xj = _sc_gather(x_lin, src_g, nin, nch)
    xj_pk = xj.reshape(e_pad * nin // 128, 128)
    msg_pk = _tc_edge(ea_pk, xj_pk, mlp_w1, mlp_w2, rep, sel, be=3200,
                      e_pad=e_pad)
    msg = msg_pk.reshape(e_pad, nout)
    parts = _sc_scatter(msg, dst_g, zeros_acc, n_acc, n, nout, nch)
    out = _tc_combine(parts, x, root, bias.reshape(1, nout), bn=2000)
    return out


# speedup vs baseline: 1.7042x; 1.0605x over previous
import functools

import jax
import jax.numpy as jnp
from jax import lax
from jax.experimental import pallas as pl
from jax.experimental.pallas import tpu as pltpu
from jax.experimental.pallas import tpu_sc as plsc

NC, NS = 2, 16
NW = NC * NS
CH = 128


def _gather_body(nch, x_hbm, idx_hbm, out_hbm, idx_v, rows_v, sem):
    c = lax.axis_index("c")
    s = lax.axis_index("s")
    wid = s * NC + c
    epw = nch * CH
    pltpu.sync_copy(idx_hbm.at[wid], idx_v)

    def fire(j, carry):
        pltpu.make_async_copy(
            x_hbm.at[idx_v.at[j]],
            rows_v.at[pl.ds(j * CH, CH)],
            sem,
        ).start()
        return carry

    def drain(j, carry):
        pltpu.make_async_copy(
            x_hbm.at[idx_v.at[0]],
            rows_v.at[pl.ds(0, CH)],
            sem,
        ).wait()
        return carry

    lax.fori_loop(0, nch, fire, 0)
    lax.fori_loop(0, nch, drain, 0)
    pltpu.sync_copy(rows_v, out_hbm.at[pl.ds(wid * epw, epw)])


def _sc_gather(x, idx_grouped, nin, nch):
    epw = nch * CH
    run = pl.kernel(
        functools.partial(_gather_body, nch),
        out_type=jax.ShapeDtypeStruct((NW * epw, nin), jnp.float32),
        mesh=plsc.VectorSubcoreMesh(core_axis_name="c", subcore_axis_name="s"),
        scratch_types=[
            pltpu.VMEM((nch, CH), jnp.int32),
            pltpu.VMEM((epw, nin), jnp.float32),
            pltpu.SemaphoreType.DMA,
        ],
        compiler_params=pltpu.CompilerParams(use_tc_tiling_on_sc=False),
    )
    return run(x, idx_grouped)


def _scatter_body(nch, n_acc, n_out, nout, msg_hbm, idx_hbm, zeros_hbm,
                  part_hbm, idx_v, rows_v, tmp_v, acc_sh, sem):
    c = lax.axis_index("c")
    s = lax.axis_index("s")
    wid = s * NC + c
    epw = nch * CH
    cnt = n_acc // NS

    pltpu.sync_copy(zeros_hbm.at[pl.ds(s * cnt, cnt)], tmp_v)
    pltpu.sync_copy(tmp_v, acc_sh.at[pl.ds(s * cnt, cnt)])

    pltpu.sync_copy(idx_hbm.at[wid], idx_v)
    pltpu.sync_copy(msg_hbm.at[pl.ds(wid * epw, epw)], rows_v)
    plsc.subcore_barrier()

    def fire(j, carry):
        pltpu.async_copy(
            rows_v.at[pl.ds(j * CH, CH)],
            acc_sh.at[idx_v.at[j]],
            sem,
            add=True,
        )
        return carry

    def drain(j, carry):
        pltpu.make_async_copy(
            rows_v.at[pl.ds(0, CH)],
            acc_sh.at[idx_v.at[0]],
            sem,
        ).wait()
        return carry

    lax.fori_loop(0, nch, fire, 0)
    lax.fori_loop(0, nch, drain, 0)
    plsc.subcore_barrier()

    ocnt = n_out // NS
    pltpu.sync_copy(acc_sh.at[pl.ds(s * ocnt, ocnt)], tmp_v.at[pl.ds(0, ocnt)])
    pltpu.sync_copy(tmp_v.at[pl.ds(0, ocnt)],
                    part_hbm.at[c].at[pl.ds(s * ocnt, ocnt)])


def _sc_scatter(msg, idx_grouped, zeros_acc, n_acc, n_out, nout, nch):
    epw = nch * CH
    run = pl.kernel(
        functools.partial(_scatter_body, nch, n_acc, n_out, nout),
        out_type=jax.ShapeDtypeStruct((NC, n_out, nout), jnp.float32),
        mesh=plsc.VectorSubcoreMesh(core_axis_name="c", subcore_axis_name="s"),
        scratch_types=[
            pltpu.VMEM((nch, CH), jnp.int32),
            pltpu.VMEM((epw, nout), jnp.float32),
            pltpu.VMEM((n_acc // NS, nout), jnp.float32),
            pltpu.VMEM_SHARED((n_acc, nout), jnp.float32),
            pltpu.SemaphoreType.DMA,
        ],
        compiler_params=pltpu.CompilerParams(use_tc_tiling_on_sc=False),
    )
    return run(msg, idx_grouped, zeros_acc)


def _edge_tc(be, nin, nout, ea_ref, xj_ref, w1_ref, w2_ref, rep_ref, sel_ref,
             msg_ref):
    g = 128 // nin
    bp = be // g
    xj = jnp.concatenate(
        [xj_ref[:, k * nin:(k + 1) * nin] for k in range(g)], axis=0)
    a = lax.dot_general(ea_ref[...], w1_ref[...], (((0,), (0,)), ((), ())),
                        preferred_element_type=jnp.float32)
    r = jnp.maximum(a, 0.0)
    h = jnp.dot(r.astype(jnp.bfloat16), w2_ref[...].astype(jnp.bfloat16),
                preferred_element_type=jnp.float32)
    xr = jnp.dot(xj, rep_ref[...], preferred_element_type=jnp.float32)
    msg = jnp.dot(xr * h, sel_ref[...], preferred_element_type=jnp.float32)
    msg_ref[...] = jnp.concatenate(
        [msg[k * bp:(k + 1) * bp, :] for k in range(g)], axis=1)


def _tc_edge(ea_t, xj_pk, w1, w2, rep, sel, be, e_pad, interpret=False):
    nin = w1.shape[0]
    hid = w2.shape[1]
    nout = sel.shape[1]
    e = ea_t.shape[1]
    grid = e // be
    bp = be * nin // 128
    bo = be * nout // 128
    return pl.pallas_call(
        functools.partial(_edge_tc, be, nin, nout),
        grid=(grid,),
        in_specs=[
            pl.BlockSpec((nin, be), lambda i: (0, i)),
            pl.BlockSpec((bp, 128), lambda i: (i, 0)),
            pl.BlockSpec((nin, hid), lambda i: (0, 0)),
            pl.BlockSpec((hid, hid), lambda i: (0, 0)),
            pl.BlockSpec((nin, hid), lambda i: (0, 0)),
            pl.BlockSpec((hid, nout), lambda i: (0, 0)),
        ],
        out_specs=pl.BlockSpec((bo, 128), lambda i: (i, 0)),
        out_shape=jax.ShapeDtypeStruct((e_pad * nout // 128, 128),
                                       jnp.float32),
        interpret=interpret,
    )(ea_t, xj_pk, w1, w2, rep, sel)


def _combine_tc(p_ref, x_ref, root_ref, bias_ref, out_ref):
    xr = jnp.dot(x_ref[...], root_ref[...], preferred_element_type=jnp.float32)
    out_ref[...] = p_ref[0] + p_ref[1] + xr + bias_ref[...]


def _tc_combine(parts, x, root, bias2d, bn, interpret=False):
    n, nin = x.shape
    nout = root.shape[1]
    grid = n // bn
    return pl.pallas_call(
        _combine_tc,
        grid=(grid,),
        in_specs=[
            pl.BlockSpec((NC, bn, nout), lambda i: (0, i, 0)),
            pl.BlockSpec((bn, nin), lambda i: (i, 0)),
            pl.BlockSpec((nin, nout), lambda i: (0, 0)),
            pl.BlockSpec((1, nout), lambda i: (0, 0)),
        ],
        out_specs=pl.BlockSpec((bn, nout), lambda i: (i, 0)),
        out_shape=jax.ShapeDtypeStruct((n, nout), jnp.float32),
        interpret=interpret,
    )(parts, x, root, bias2d)


def kernel(x, edge_index, edge_attr, mlp_w1, mlp_w2, root, bias):
    n, nin = x.shape
    e = edge_index.shape[1]
    hid = mlp_w1.shape[1]
    nout = root.shape[1]
    assert n % NS == 0

    nch = -(-e // (NW * CH))
    e_pad = NW * nch * CH
    g = 128 // nin
    be = 3200
    bp = be // g
    perm = lambda v: v.reshape(e // be, g, bp).transpose(0, 2, 1).reshape(e)
    src = perm(edge_index[0])
    dst = perm(edge_index[1])
    pad = e_pad - e
    if pad:
        src = jnp.concatenate([src, jnp.zeros((pad,), jnp.int32)])
        dst = jnp.concatenate([dst, jnp.full((pad,), n, jnp.int32)])
    src_g = src.reshape(NW, nch, CH)
    dst_g = dst.reshape(NW, nch, CH)
    ea_t = edge_attr.T

    n_acc = -(-(n + 1) // NS) * NS
    zeros_acc = jnp.zeros((n_acc * nout // 128, 128),
                          jnp.float32).reshape(n_acc, nout)

    x_pk = lax.optimization_barrier(x.reshape(n * nin // 128, 128))
    x_lin = x_pk.reshape(n, nin)

    ii = lax.broadcasted_iota(jnp.int32, (nin, hid), 0)
    cc = lax.broadcasted_iota(jnp.int32, (nin, hid), 1)
    rep = (cc // nout == ii).astype(jnp.float32)
    c2 = lax.broadcasted_iota(jnp.int32, (hid, nout), 0)
    oo = lax.broadcasted_iota(jnp.int32, (hid, nout), 1)
    sel = (c2 % nout == oo).astype(jnp.float32)

    xj = _sc_gather(x_lin, src_g, nin, nch)
    xj_pk = xj.reshape(e_pad * nin // 128, 128)
    msg_pk = _tc_edge(ea_t, xj_pk, mlp_w1, mlp_w2, rep, sel, be=be,
                      e_pad=e_pad)
    msg = msg_pk.reshape(e_pad, nout)
    parts = _sc_scatter(msg, dst_g, zeros_acc, n_acc, n, nout, nch)
    out = _tc_combine(parts, x, root, bias.reshape(1, nout), bn=2000)
    return out


# speedup vs baseline: 1.9114x; 1.1216x over previous
import functools

import jax
import jax.numpy as jnp
from jax import lax
from jax.experimental import pallas as pl
from jax.experimental.pallas import tpu as pltpu
from jax.experimental.pallas import tpu_sc as plsc

NC, NS = 2, 16
NW = NC * NS
CH = 128


def _gather_body(nch, x_hbm, idx_hbm, out_hbm, idx_v, rows_v, sem):
    c = lax.axis_index("c")
    s = lax.axis_index("s")
    wid = s * NC + c
    epw = nch * CH
    pltpu.sync_copy(idx_hbm.at[wid], idx_v)

    def fire(j, carry):
        pltpu.make_async_copy(
            x_hbm.at[idx_v.at[j]],
            rows_v.at[pl.ds(j * CH, CH)],
            sem,
        ).start()
        return carry

    def drain(j, carry):
        pltpu.make_async_copy(
            x_hbm.at[idx_v.at[0]],
            rows_v.at[pl.ds(0, CH)],
            sem,
        ).wait()
        return carry

    lax.fori_loop(0, nch, fire, 0)
    lax.fori_loop(0, nch, drain, 0)
    pltpu.sync_copy(rows_v, out_hbm.at[pl.ds(wid * epw, epw)])


def _sc_gather(x, idx_grouped, nin, nch):
    epw = nch * CH
    run = pl.kernel(
        functools.partial(_gather_body, nch),
        out_type=jax.ShapeDtypeStruct((NW * epw, nin), jnp.float32),
        mesh=plsc.VectorSubcoreMesh(core_axis_name="c", subcore_axis_name="s"),
        scratch_types=[
            pltpu.VMEM((nch, CH), jnp.int32),
            pltpu.VMEM((epw, nin), jnp.float32),
            pltpu.SemaphoreType.DMA,
        ],
        compiler_params=pltpu.CompilerParams(use_tc_tiling_on_sc=False),
    )
    return run(x, idx_grouped)


def _scatter_body(nch, n_acc, n_out, nout, msg_hbm, idx_hbm, zeros_hbm,
                  part_hbm, idx_v, rows_v, tmp_v, acc_sh, sem):
    c = lax.axis_index("c")
    s = lax.axis_index("s")
    wid = s * NC + c
    epw = nch * CH
    cnt = n_acc // NS

    pltpu.sync_copy(zeros_hbm.at[pl.ds(s * cnt, cnt)], tmp_v)
    pltpu.sync_copy(tmp_v, acc_sh.at[pl.ds(s * cnt, cnt)])

    pltpu.sync_copy(idx_hbm.at[wid], idx_v)
    pltpu.sync_copy(msg_hbm.at[pl.ds(wid * epw, epw)], rows_v)
    plsc.subcore_barrier()

    def fire(j, carry):
        pltpu.async_copy(
            rows_v.at[pl.ds(j * CH, CH)],
            acc_sh.at[idx_v.at[j]],
            sem,
            add=True,
        )
        return carry

    def drain(j, carry):
        pltpu.make_async_copy(
            rows_v.at[pl.ds(0, CH)],
            acc_sh.at[idx_v.at[0]],
            sem,
        ).wait()
        return carry

    lax.fori_loop(0, nch, fire, 0)
    lax.fori_loop(0, nch, drain, 0)
    plsc.subcore_barrier()

    ocnt = n_out // NS
    pltpu.sync_copy(acc_sh.at[pl.ds(s * ocnt, ocnt)], tmp_v.at[pl.ds(0, ocnt)])
    pltpu.sync_copy(tmp_v.at[pl.ds(0, ocnt)],
                    part_hbm.at[c].at[pl.ds(s * ocnt, ocnt)])


def _sc_scatter(msg, idx_grouped, zeros_acc, n_acc, n_out, nout, nch):
    epw = nch * CH
    run = pl.kernel(
        functools.partial(_scatter_body, nch, n_acc, n_out, nout),
        out_type=jax.ShapeDtypeStruct((NC, n_out, nout), jnp.float32),
        mesh=plsc.VectorSubcoreMesh(core_axis_name="c", subcore_axis_name="s"),
        scratch_types=[
            pltpu.VMEM((nch, CH), jnp.int32),
            pltpu.VMEM((epw, nout), jnp.float32),
            pltpu.VMEM((n_acc // NS, nout), jnp.float32),
            pltpu.VMEM_SHARED((n_acc, nout), jnp.float32),
            pltpu.SemaphoreType.DMA,
        ],
        compiler_params=pltpu.CompilerParams(use_tc_tiling_on_sc=False),
    )
    return run(msg, idx_grouped, zeros_acc)


def _edge_tc(be, nin, nout, ea_ref, xj_ref, w1_ref, w2_ref, rep_ref, sel_ref,
             msg_ref):
    g = 128 // nin
    bp = be // g
    xj = jnp.concatenate(
        [xj_ref[:, k * nin:(k + 1) * nin] for k in range(g)], axis=0)
    a = lax.dot_general(ea_ref[...], w1_ref[...], (((0,), (0,)), ((), ())),
                        preferred_element_type=jnp.float32)
    r = jnp.maximum(a, 0.0)
    h = jnp.dot(r.astype(jnp.bfloat16), w2_ref[...].astype(jnp.bfloat16),
                preferred_element_type=jnp.float32)
    xr = jnp.dot(xj, rep_ref[...], preferred_element_type=jnp.float32)
    msg = jnp.dot(xr * h, sel_ref[...], preferred_element_type=jnp.float32)
    msg_ref[...] = jnp.concatenate(
        [msg[k * bp:(k + 1) * bp, :] for k in range(g)], axis=1)


def _tc_edge(ea_t, xj_pk, w1, w2, rep, sel, be, e_h, e_pad, blk_off=0,
             interpret=False):
    nin = w1.shape[0]
    hid = w2.shape[1]
    nout = sel.shape[1]
    grid = e_h // be
    bp = be * nin // 128
    bo = be * nout // 128
    return pl.pallas_call(
        functools.partial(_edge_tc, be, nin, nout),
        grid=(grid,),
        in_specs=[
            pl.BlockSpec((nin, be), lambda i: (0, i + blk_off)),
            pl.BlockSpec((bp, 128), lambda i: (i, 0)),
            pl.BlockSpec((nin, hid), lambda i: (0, 0)),
            pl.BlockSpec((hid, hid), lambda i: (0, 0)),
            pl.BlockSpec((nin, hid), lambda i: (0, 0)),
            pl.BlockSpec((hid, nout), lambda i: (0, 0)),
        ],
        out_specs=pl.BlockSpec((bo, 128), lambda i: (i, 0)),
        out_shape=jax.ShapeDtypeStruct((e_pad * nout // 128, 128),
                                       jnp.float32),
        interpret=interpret,
    )(ea_t, xj_pk, w1, w2, rep, sel)


def _combine_tc(nparts, *refs):
    p_refs = refs[:nparts]
    x_ref, root_ref, bias_ref, out_ref = refs[nparts:]
    xr = jnp.dot(x_ref[...], root_ref[...], preferred_element_type=jnp.float32)
    acc = xr + bias_ref[...]
    for p in p_refs:
        acc = acc + p[0] + p[1]
    out_ref[...] = acc


def _tc_combine(parts_list, x, root, bias2d, bn, interpret=False):
    n, nin = x.shape
    nout = root.shape[1]
    grid = n // bn
    return pl.pallas_call(
        functools.partial(_combine_tc, len(parts_list)),
        grid=(grid,),
        in_specs=[pl.BlockSpec((NC, bn, nout), lambda i: (0, i, 0))
                  for _ in parts_list] + [
            pl.BlockSpec((bn, nin), lambda i: (i, 0)),
            pl.BlockSpec((nin, nout), lambda i: (0, 0)),
            pl.BlockSpec((1, nout), lambda i: (0, 0)),
        ],
        out_specs=pl.BlockSpec((bn, nout), lambda i: (i, 0)),
        out_shape=jax.ShapeDtypeStruct((n, nout), jnp.float32),
        interpret=interpret,
    )(*parts_list, x, root, bias2d)


def kernel(x, edge_index, edge_attr, mlp_w1, mlp_w2, root, bias):
    n, nin = x.shape
    e = edge_index.shape[1]
    hid = mlp_w1.shape[1]
    nout = root.shape[1]
    assert n % NS == 0

    nsplit = 2
    e_h = e // nsplit
    nch = -(-e_h // (NW * CH))
    e_pad = NW * nch * CH
    g = 128 // nin
    be = 3200
    bp = be // g
    assert e_h % be == 0
    pad = e_pad - e_h

    def prep_idx(v):
        return v.reshape(e_h // be, g, bp).transpose(0, 2, 1).reshape(e_h)

    src_gs, dst_gs = [], []
    for hh in range(nsplit):
        sl = slice(hh * e_h, (hh + 1) * e_h)
        sh = prep_idx(edge_index[0][sl])
        dh = prep_idx(edge_index[1][sl])
        if pad:
            sh = jnp.concatenate([sh, jnp.zeros((pad,), jnp.int32)])
            dh = jnp.concatenate([dh, jnp.full((pad,), n, jnp.int32)])
        src_gs.append(sh.reshape(NW, nch, CH))
        dst_gs.append(dh.reshape(NW, nch, CH))
    ea_t = edge_attr.T

    n_acc = -(-(n + 1) // NS) * NS
    zeros_acc = jnp.zeros((n_acc * nout // 128, 128),
                          jnp.float32).reshape(n_acc, nout)

    x_pk = lax.optimization_barrier(x.reshape(n * nin // 128, 128))
    x_lin = x_pk.reshape(n, nin)

    ii = lax.broadcasted_iota(jnp.int32, (nin, hid), 0)
    cc = lax.broadcasted_iota(jnp.int32, (nin, hid), 1)
    rep = (cc // nout == ii).astype(jnp.float32)
    c2 = lax.broadcasted_iota(jnp.int32, (hid, nout), 0)
    oo = lax.broadcasted_iota(jnp.int32, (hid, nout), 1)
    sel = (c2 % nout == oo).astype(jnp.float32)

    parts_list = []
    for hh in range(nsplit):
        xj = _sc_gather(x_lin, src_gs[hh], nin, nch)
        xj_pk = xj.reshape(e_pad * nin // 128, 128)
        msg_pk = _tc_edge(ea_t, xj_pk, mlp_w1, mlp_w2, rep, sel,
                          be=be, e_h=e_h, e_pad=e_pad,
                          blk_off=hh * (e_h // be))
        msg = msg_pk.reshape(e_pad, nout)
        parts_list.append(
            _sc_scatter(msg, dst_gs[hh], zeros_acc, n_acc, n, nout, nch))
    out = _tc_combine(parts_list, x, root, bias.reshape(1, nout), bn=2000)
    return out


# speedup vs baseline: 1.9631x; 1.0270x over previous
import functools

import jax
import jax.numpy as jnp
from jax import lax
from jax.experimental import pallas as pl
from jax.experimental.pallas import tpu as pltpu
from jax.experimental.pallas import tpu_sc as plsc

NC, NS = 2, 16
NW = NC * NS
CH = 128


def _gather_body(nch, x_hbm, idx_hbm, out_hbm, idx_v, rows_v, sem):
    c = lax.axis_index("c")
    s = lax.axis_index("s")
    wid = s * NC + c
    epw = nch * CH
    pltpu.sync_copy(idx_hbm.at[wid], idx_v)

    def fire(j, carry):
        pltpu.make_async_copy(
            x_hbm.at[idx_v.at[j]],
            rows_v.at[pl.ds(j * CH, CH)],
            sem,
        ).start()
        return carry

    def drain(j, carry):
        pltpu.make_async_copy(
            x_hbm.at[idx_v.at[0]],
            rows_v.at[pl.ds(0, CH)],
            sem,
        ).wait()
        return carry

    lax.fori_loop(0, nch, fire, 0)
    lax.fori_loop(0, nch, drain, 0)
    pltpu.sync_copy(rows_v, out_hbm.at[pl.ds(wid * epw, epw)])


def _sc_gather(x, idx_grouped, nin, nch):
    epw = nch * CH
    run = pl.kernel(
        functools.partial(_gather_body, nch),
        out_type=jax.ShapeDtypeStruct((NW * epw, nin), jnp.float32),
        mesh=plsc.VectorSubcoreMesh(core_axis_name="c", subcore_axis_name="s"),
        scratch_types=[
            pltpu.VMEM((nch, CH), jnp.int32),
            pltpu.VMEM((epw, nin), jnp.float32),
            pltpu.SemaphoreType.DMA,
        ],
        compiler_params=pltpu.CompilerParams(use_tc_tiling_on_sc=False),
    )
    return run(x, idx_grouped)


def _scatter_body(nch, n_acc, n_out, nout, msg_hbm, idx_hbm, zeros_hbm,
                  part_hbm, idx_v, rows_v, tmp_v, acc_sh, sem):
    c = lax.axis_index("c")
    s = lax.axis_index("s")
    wid = s * NC + c
    epw = nch * CH
    cnt = n_acc // NS

    pltpu.sync_copy(zeros_hbm.at[pl.ds(s * cnt, cnt)], tmp_v)
    pltpu.sync_copy(tmp_v, acc_sh.at[pl.ds(s * cnt, cnt)])

    pltpu.sync_copy(idx_hbm.at[wid], idx_v)
    pltpu.sync_copy(msg_hbm.at[pl.ds(wid * epw, epw)], rows_v)
    plsc.subcore_barrier()

    def fire(j, carry):
        pltpu.async_copy(
            rows_v.at[pl.ds(j * CH, CH)],
            acc_sh.at[idx_v.at[j]],
            sem,
            add=True,
        )
        return carry

    def drain(j, carry):
        pltpu.make_async_copy(
            rows_v.at[pl.ds(0, CH)],
            acc_sh.at[idx_v.at[0]],
            sem,
        ).wait()
        return carry

    lax.fori_loop(0, nch, fire, 0)
    lax.fori_loop(0, nch, drain, 0)
    plsc.subcore_barrier()

    ocnt = n_out // NS
    pltpu.sync_copy(acc_sh.at[pl.ds(s * ocnt, ocnt)], tmp_v.at[pl.ds(0, ocnt)])
    pltpu.sync_copy(tmp_v.at[pl.ds(0, ocnt)],
                    part_hbm.at[c].at[pl.ds(s * ocnt, ocnt)])


def _sc_scatter(msg, idx_grouped, zeros_acc, n_acc, n_out, nout, nch):
    epw = nch * CH
    run = pl.kernel(
        functools.partial(_scatter_body, nch, n_acc, n_out, nout),
        out_type=jax.ShapeDtypeStruct((NC, n_out, nout), jnp.float32),
        mesh=plsc.VectorSubcoreMesh(core_axis_name="c", subcore_axis_name="s"),
        scratch_types=[
            pltpu.VMEM((nch, CH), jnp.int32),
            pltpu.VMEM((epw, nout), jnp.float32),
            pltpu.VMEM((n_acc // NS, nout), jnp.float32),
            pltpu.VMEM_SHARED((n_acc, nout), jnp.float32),
            pltpu.SemaphoreType.DMA,
        ],
        compiler_params=pltpu.CompilerParams(use_tc_tiling_on_sc=False),
    )
    return run(msg, idx_grouped, zeros_acc)


def _edge_tc(be, nin, nout, ea_ref, xj_ref, w1_ref, w2_ref, rep_ref, sel_ref,
             msg_ref):
    g = 128 // nin
    bp = be // g
    xj = jnp.concatenate(
        [xj_ref[:, k * nin:(k + 1) * nin] for k in range(g)], axis=0)
    a = lax.dot_general(ea_ref[...], w1_ref[...], (((0,), (0,)), ((), ())),
                        preferred_element_type=jnp.float32)
    r = jnp.maximum(a, 0.0)
    h = jnp.dot(r.astype(jnp.bfloat16), w2_ref[...].astype(jnp.bfloat16),
                preferred_element_type=jnp.float32)
    xr = jnp.dot(xj, rep_ref[...], preferred_element_type=jnp.float32)
    msg = jnp.dot(xr * h, sel_ref[...], preferred_element_type=jnp.float32)
    msg_ref[...] = jnp.concatenate(
        [msg[k * bp:(k + 1) * bp, :] for k in range(g)], axis=1)


def _tc_edge(ea_t, xj_pk, w1, w2, rep, sel, be, e_h, e_pad, blk_off=0,
             interpret=False):
    nin = w1.shape[0]
    hid = w2.shape[1]
    nout = sel.shape[1]
    grid = e_h // be
    bp = be * nin // 128
    bo = be * nout // 128
    return pl.pallas_call(
        functools.partial(_edge_tc, be, nin, nout),
        grid=(grid,),
        in_specs=[
            pl.BlockSpec((nin, be), lambda i: (0, i + blk_off)),
            pl.BlockSpec((bp, 128), lambda i: (i, 0)),
            pl.BlockSpec((nin, hid), lambda i: (0, 0)),
            pl.BlockSpec((hid, hid), lambda i: (0, 0)),
            pl.BlockSpec((nin, hid), lambda i: (0, 0)),
            pl.BlockSpec((hid, nout), lambda i: (0, 0)),
        ],
        out_specs=pl.BlockSpec((bo, 128), lambda i: (i, 0)),
        out_shape=jax.ShapeDtypeStruct((e_pad * nout // 128, 128),
                                       jnp.float32),
        interpret=interpret,
    )(ea_t, xj_pk, w1, w2, rep, sel)


def _combine_tc(nparts, nin, nout, *refs):
    g = 128 // nout
    p_refs = refs[:nparts]
    x_ref, root_ref, bias_ref, out_ref = refs[nparts:]

    def unpack(ref2d):
        return jnp.concatenate(
            [ref2d[:, k * nin:(k + 1) * nin] for k in range(g)], axis=0)

    xv = unpack(x_ref)
    xr = jnp.dot(xv, root_ref[...], preferred_element_type=jnp.float32)
    acc = xr + bias_ref[...]
    for p in p_refs:
        acc = acc + unpack(p[0]) + unpack(p[1])
    bp = acc.shape[0] // g
    out_ref[...] = jnp.concatenate(
        [acc[k * bp:(k + 1) * bp, :] for k in range(g)], axis=1)


def _tc_combine(parts_pk_list, x_pk, root, bias2d, interpret=False):
    npk = x_pk.shape[0]
    nin = root.shape[0]
    nout = root.shape[1]
    return pl.pallas_call(
        functools.partial(_combine_tc, len(parts_pk_list), nin, nout),
        in_specs=[pl.BlockSpec((NC, npk, 128), lambda: (0, 0, 0))
                  for _ in parts_pk_list] + [
            pl.BlockSpec((npk, 128), lambda: (0, 0)),
            pl.BlockSpec((nin, nout), lambda: (0, 0)),
            pl.BlockSpec((1, nout), lambda: (0, 0)),
        ],
        out_specs=pl.BlockSpec((npk, 128), lambda: (0, 0)),
        out_shape=jax.ShapeDtypeStruct((npk, 128), jnp.float32),
        interpret=interpret,
    )(*parts_pk_list, x_pk, root, bias2d)


def kernel(x, edge_index, edge_attr, mlp_w1, mlp_w2, root, bias):
    n, nin = x.shape
    e = edge_index.shape[1]
    hid = mlp_w1.shape[1]
    nout = root.shape[1]
    assert n % NS == 0

    nsplit = 2
    e_h = e // nsplit
    nch = -(-e_h // (NW * CH))
    e_pad = NW * nch * CH
    g = 128 // nin
    be = 3200
    bp = be // g
    assert e_h % be == 0
    pad = e_pad - e_h

    def prep_idx(v):
        return v.reshape(e_h // be, g, bp).transpose(0, 2, 1).reshape(e_h)

    src_gs, dst_gs = [], []
    for hh in range(nsplit):
        sl = slice(hh * e_h, (hh + 1) * e_h)
        sh = prep_idx(edge_index[0][sl])
        dh = prep_idx(edge_index[1][sl])
        if pad:
            sh = jnp.concatenate([sh, jnp.zeros((pad,), jnp.int32)])
            dh = jnp.concatenate([dh, jnp.full((pad,), n, jnp.int32)])
        src_gs.append(sh.reshape(NW, nch, CH))
        dst_gs.append(dh.reshape(NW, nch, CH))
    ea_t = edge_attr.T

    n_acc = -(-(n + 1) // NS) * NS
    zeros_acc = jnp.zeros((n_acc * nout // 128, 128),
                          jnp.float32).reshape(n_acc, nout)

    x_pk = lax.optimization_barrier(x.reshape(n * nin // 128, 128))
    x_lin = x_pk.reshape(n, nin)

    ii = lax.broadcasted_iota(jnp.int32, (nin, hid), 0)
    cc = lax.broadcasted_iota(jnp.int32, (nin, hid), 1)
    rep = (cc // nout == ii).astype(jnp.float32)
    c2 = lax.broadcasted_iota(jnp.int32, (hid, nout), 0)
    oo = lax.broadcasted_iota(jnp.int32, (hid, nout), 1)
    sel = (c2 % nout == oo).astype(jnp.float32)

    parts_list = []
    for hh in range(nsplit):
        xj = _sc_gather(x_lin, src_gs[hh], nin, nch)
        xj_pk = xj.reshape(e_pad * nin // 128, 128)
        msg_pk = _tc_edge(ea_t, xj_pk, mlp_w1, mlp_w2, rep, sel,
                          be=be, e_h=e_h, e_pad=e_pad,
                          blk_off=hh * (e_h // be))
        msg = msg_pk.reshape(e_pad, nout)
        parts = _sc_scatter(msg, dst_gs[hh], zeros_acc, n_acc, n, nout, nch)
        parts_list.append(parts.reshape(NC, n * nout // 128, 128))
    out_pk = _tc_combine(parts_list, x_pk, root, bias.reshape(1, nout))
    return out_pk.reshape(n, nout)
